# Initial kernel scaffold; baseline (speedup 1.0000x reference)
import jax
import jax.numpy as jnp
from jax.experimental import pallas as pl


def kernel(inputs, token_table, pos_table):
    raise NotImplementedError("write your pallas kernel here")


# speedup vs baseline: 4.2890x; 4.2890x over previous
import functools

import jax
import jax.numpy as jnp
from jax import lax
from jax.experimental import pallas as pl
from jax.experimental.pallas import tpu as pltpu
from jax.experimental.pallas import tpu_sc as plsc

BATCH = 1024
SEQ = 200
D = 128
NC = 2
NS = 16
NW = NC * NS
ROWS = BATCH * SEQ
RPW = ROWS // NW
SEQ_PER_W = RPW // SEQ
HALF = SEQ // 2


def _tpe_kernel(idx_hbm, tok_hbm, pos_hbm, out_hbm, idx_v, pos_v, buf_v, sem):
    wid = lax.axis_index("s") * NC + lax.axis_index("c")
    base = wid * RPW

    pltpu.sync_copy(idx_hbm.at[wid], idx_v)
    pltpu.sync_copy(pos_hbm, pos_v)

    def seq_step(s, carry):
        cp0 = pltpu.async_copy(
            tok_hbm.at[idx_v.at[2 * s]], buf_v.at[pl.ds(0, HALF)], sem)
        cp1 = pltpu.async_copy(
            tok_hbm.at[idx_v.at[2 * s + 1]], buf_v.at[pl.ds(HALF, HALF)], sem)
        cp0.wait()
        cp1.wait()

        def row_step(r, c):
            for j in range(D // 16):
                sl = pl.ds(j * 16, 16)
                buf_v[r, sl] = buf_v[r, sl] + pos_v[r, sl]
            return c

        lax.fori_loop(0, SEQ, row_step, 0)

        pltpu.sync_copy(buf_v, out_hbm.at[pl.ds(base + s * SEQ, SEQ)])
        return carry

    lax.fori_loop(0, SEQ_PER_W, seq_step, 0)


@jax.jit
def kernel(inputs, token_table, pos_table):
    idx = inputs.reshape(NW, 2 * SEQ_PER_W, HALF).astype(jnp.int32)
    run = pl.kernel(
        _tpe_kernel,
        out_type=jax.ShapeDtypeStruct((ROWS, D), jnp.float32),
        mesh=plsc.VectorSubcoreMesh(core_axis_name="c", subcore_axis_name="s"),
        scratch_types=[
            pltpu.VMEM((2 * SEQ_PER_W, HALF), jnp.int32),
            pltpu.VMEM((SEQ, D), jnp.float32),
            pltpu.VMEM((SEQ, D), jnp.float32),
            pltpu.SemaphoreType.DMA,
        ],
    )
    out = run(idx, token_table, pos_table)
    return out.reshape(BATCH, SEQ, D)


# speedup vs baseline: 6.3302x; 1.4759x over previous
import functools

import jax
import jax.numpy as jnp
from jax import lax
from jax.experimental import pallas as pl
from jax.experimental.pallas import tpu as pltpu
from jax.experimental.pallas import tpu_sc as plsc

BATCH = 1024
SEQ = 200
D = 128
NC = 2
NS = 16
NW = NC * NS
ROWS = BATCH * SEQ
RPW = ROWS // NW
SEQ_PER_W = RPW // SEQ
HALF = SEQ // 2
NBUF = 3


def _tpe_kernel(idx_hbm, tok_hbm, pos_hbm, out_hbm,
                idx_v, pos_v, b0, b1, b2, g0, g1, g2, s0, s1, s2):
    bufs = (b0, b1, b2)
    gsems = (g0, g1, g2)
    ssems = (s0, s1, s2)
    wid = lax.axis_index("s") * NC + lax.axis_index("c")
    base = wid * RPW

    pltpu.sync_copy(idx_hbm.at[wid], idx_v)
    pltpu.sync_copy(pos_hbm, pos_v)

    def issue_gather(s):
        b = bufs[s % NBUF]
        sem = gsems[s % NBUF]
        c0 = pltpu.async_copy(tok_hbm.at[idx_v.at[2 * s]],
                              b.at[pl.ds(0, HALF)], sem)
        c1 = pltpu.async_copy(tok_hbm.at[idx_v.at[2 * s + 1]],
                              b.at[pl.ds(HALF, HALF)], sem)
        return (c0, c1)

    def issue_store(s):
        b = bufs[s % NBUF]
        return pltpu.async_copy(b, out_hbm.at[pl.ds(base + s * SEQ, SEQ)],
                                ssems[s % NBUF])

    gh = {}
    sh = {}
    for s in range(NBUF):
        gh[s] = issue_gather(s)

    for s in range(SEQ_PER_W):
        if s >= 1 and s + 2 < SEQ_PER_W:
            sh[s - 1].wait()
            gh[s + 2] = issue_gather(s + 2)
        for c in gh[s]:
            c.wait()
        b = bufs[s % NBUF]

        def row_step(r, carry):
            for j in range(D // 16):
                sl = pl.ds(j * 16, 16)
                plsc.addupdate(b.at[r, sl], pos_v[r, sl])
            return carry

        lax.fori_loop(0, SEQ, row_step, 0)
        sh[s] = issue_store(s)

    for s in range(SEQ_PER_W - NBUF, SEQ_PER_W):
        sh[s].wait()


@jax.jit
def kernel(inputs, token_table, pos_table):
    idx = inputs.reshape(NW, 2 * SEQ_PER_W, HALF).astype(jnp.int32)
    run = pl.kernel(
        _tpe_kernel,
        out_type=jax.ShapeDtypeStruct((ROWS, D), jnp.float32),
        mesh=plsc.VectorSubcoreMesh(core_axis_name="c", subcore_axis_name="s"),
        scratch_types=[
            pltpu.VMEM((2 * SEQ_PER_W, HALF), jnp.int32),
            pltpu.VMEM((SEQ, D), jnp.float32),
            pltpu.VMEM((SEQ, D), jnp.float32),
            pltpu.VMEM((SEQ, D), jnp.float32),
            pltpu.VMEM((SEQ, D), jnp.float32),
            pltpu.SemaphoreType.DMA,
            pltpu.SemaphoreType.DMA,
            pltpu.SemaphoreType.DMA,
            pltpu.SemaphoreType.DMA,
            pltpu.SemaphoreType.DMA,
            pltpu.SemaphoreType.DMA,
        ],
    )
    out = run(idx, token_table, pos_table)
    return out.reshape(BATCH, SEQ, D)
